# Initial kernel scaffold; baseline (speedup 1.0000x reference)
import jax
import jax.numpy as jnp
from jax.experimental import pallas as pl


def kernel(batch, edge_index, emb, W1, b1, W2, b2):
    raise NotImplementedError("write your pallas kernel here")


# speedup vs baseline: 5.6865x; 5.6865x over previous
import functools

import jax
import jax.numpy as jnp
from jax import lax
from jax.experimental import pallas as pl
from jax.experimental.pallas import tpu as pltpu
from jax.experimental.pallas import tpu_sc as plsc

N_NODES = 10000
D = 128
E = 320000
BATCH = 4096

NC, NS = 2, 16
NW = NC * NS
EPW = E // NW
K = 80
CW = EPW // K
BPW = BATCH // NW
BPT = BATCH // NS
N_PAD = 10240
ZR = N_PAD // NS


def _tc1_body(x_ref, w_ref, b_ref, a_ref, bb_ref):
    x = x_ref[...]
    a_ref[...] = jnp.dot(x, w_ref[:D, :], preferred_element_type=jnp.float32)
    bb_ref[...] = (
        jnp.dot(x, w_ref[D:, :], preferred_element_type=jnp.float32) + b_ref[...]
    )


def _precompute_ab(x, W1, b1):
    blk = N_NODES // 10
    return pl.pallas_call(
        _tc1_body,
        grid=(N_NODES // blk,),
        in_specs=[
            pl.BlockSpec((blk, D), lambda i: (i, 0)),
            pl.BlockSpec((2 * D, D), lambda i: (0, 0)),
            pl.BlockSpec((1, D), lambda i: (0, 0)),
        ],
        out_specs=[
            pl.BlockSpec((blk, D), lambda i: (i, 0)),
            pl.BlockSpec((blk, D), lambda i: (i, 0)),
        ],
        out_shape=[
            jax.ShapeDtypeStruct((N_NODES, D), jnp.float32),
            jax.ShapeDtypeStruct((N_NODES, D), jnp.float32),
        ],
    )(x, W1, b1.reshape(1, D))


def _sc_body(src_hbm, dst_hbm, emb_hbm, a_hbm, b_hbm, batch_hbm,
             xb_out, aggb_out,
             srcv, dstv, arows, brows, bidxv, grows, aggsh, sem1, sem2):
    cid = lax.axis_index("c")
    sid = lax.axis_index("s")
    wid = sid * NC + cid

    zero16 = jnp.zeros((16,), jnp.float32)

    def zrow(i, carry):
        for j in range(D // 16):
            grows[i, pl.ds(j * 16, 16)] = zero16
        return carry

    lax.fori_loop(0, BPW, zrow, None)
    for r in range(ZR // BPW):
        pltpu.sync_copy(grows, aggsh.at[pl.ds(sid * ZR + r * BPW, BPW)])
    plsc.subcore_barrier()

    def edge_chunk(c, carry):
        row = wid * CW + c
        pltpu.sync_copy(src_hbm.at[row], srcv)
        pltpu.sync_copy(dst_hbm.at[row], dstv)
        cpa = pltpu.async_copy(a_hbm.at[srcv], arows, sem1)
        cpb = pltpu.async_copy(b_hbm.at[dstv], brows, sem2)
        cpa.wait()
        cpb.wait()

        def fuse(i, inner):
            for j in range(D // 16):
                s = pl.ds(j * 16, 16)
                arows[i, s] = jnp.maximum(arows[i, s] + brows[i, s], 0.0)
            return inner

        lax.fori_loop(0, K, fuse, None)
        pltpu.sync_copy(arows, aggsh.at[dstv], add=True)
        return carry

    lax.fori_loop(0, CW, edge_chunk, None)
    plsc.subcore_barrier()

    base = wid * BPW
    pltpu.sync_copy(batch_hbm.at[pl.ds(base, BPW)], bidxv)
    pltpu.async_copy(emb_hbm.at[bidxv], grows, sem1).wait()
    pltpu.sync_copy(grows, xb_out.at[pl.ds(base, BPW)])

    for r in range(BPT // BPW):
        b0 = sid * BPT + r * BPW
        pltpu.sync_copy(batch_hbm.at[pl.ds(b0, BPW)], bidxv)
        pltpu.async_copy(aggsh.at[bidxv], grows, sem2).wait()
        pltpu.sync_copy(grows, aggb_out.at[cid, pl.ds(b0, BPW)])


_sc_call = pl.kernel(
    _sc_body,
    out_type=(
        jax.ShapeDtypeStruct((BATCH, D), jnp.float32),
        jax.ShapeDtypeStruct((NC, BATCH, D), jnp.float32),
    ),
    mesh=plsc.VectorSubcoreMesh(core_axis_name="c", subcore_axis_name="s"),
    scratch_types=[
        pltpu.VMEM((K,), jnp.int32),
        pltpu.VMEM((K,), jnp.int32),
        pltpu.VMEM((K, D), jnp.float32),
        pltpu.VMEM((K, D), jnp.float32),
        pltpu.VMEM((BPW,), jnp.int32),
        pltpu.VMEM((BPW, D), jnp.float32),
        pltpu.VMEM_SHARED((N_PAD, D), jnp.float32),
        pltpu.SemaphoreType.DMA,
        pltpu.SemaphoreType.DMA,
    ],
)


def _tc2_body(xb_ref, a0_ref, a1_ref, w_ref, b_ref, o_ref):
    o_ref[...] = (
        jnp.dot(xb_ref[...], w_ref[:D, :], preferred_element_type=jnp.float32)
        + jnp.dot(a0_ref[...] + a1_ref[...], w_ref[D:, :],
                  preferred_element_type=jnp.float32)
        + b_ref[...]
    )


def _final(xb, a0, a1, W2, b2):
    blk = 1024
    return pl.pallas_call(
        _tc2_body,
        grid=(BATCH // blk,),
        in_specs=[
            pl.BlockSpec((blk, D), lambda i: (i, 0)),
            pl.BlockSpec((blk, D), lambda i: (i, 0)),
            pl.BlockSpec((blk, D), lambda i: (i, 0)),
            pl.BlockSpec((2 * D, D), lambda i: (0, 0)),
            pl.BlockSpec((1, D), lambda i: (0, 0)),
        ],
        out_specs=pl.BlockSpec((blk, D), lambda i: (i, 0)),
        out_shape=jax.ShapeDtypeStruct((BATCH, D), jnp.float32),
    )(xb, a0, a1, W2, b2.reshape(1, D))


def kernel(batch, edge_index, emb, W1, b1, W2, b2):
    a, bb = _precompute_ab(emb, W1, b1)
    src = edge_index[0].reshape(NW * CW, K)
    dst = edge_index[1].reshape(NW * CW, K)
    xb, aggb = _sc_call(src, dst, emb, a, bb, batch)
    return _final(xb, aggb[0], aggb[1], W2, b2)


# speedup vs baseline: 10.8867x; 1.9145x over previous
import functools

import jax
import jax.numpy as jnp
from jax import lax
from jax.experimental import pallas as pl
from jax.experimental.pallas import tpu as pltpu
from jax.experimental.pallas import tpu_sc as plsc

N_NODES = 10000
D = 128
E = 320000
BATCH = 4096

NC, NS = 2, 16
NW = NC * NS
EPW = E // NW
K = 128
BPW = BATCH // NW
BPT = BATCH // NS
GARBAGE = BATCH
ACC = 4224
ZR = ACC // NS
VL = 16
NMARK = N_NODES + VL


def _tc1_body(x_ref, w_ref, b_ref, a_ref, bb_ref):
    x = x_ref[...]
    a_ref[...] = jnp.dot(x, w_ref[:D, :], preferred_element_type=jnp.float32)
    bb_ref[...] = (
        jnp.dot(x, w_ref[D:, :], preferred_element_type=jnp.float32) + b_ref[...]
    )


def _precompute_ab(x, W1, b1):
    blk = N_NODES // 10
    return pl.pallas_call(
        _tc1_body,
        grid=(N_NODES // blk,),
        in_specs=[
            pl.BlockSpec((blk, D), lambda i: (i, 0)),
            pl.BlockSpec((2 * D, D), lambda i: (0, 0)),
            pl.BlockSpec((1, D), lambda i: (0, 0)),
        ],
        out_specs=[
            pl.BlockSpec((blk, D), lambda i: (i, 0)),
            pl.BlockSpec((blk, D), lambda i: (i, 0)),
        ],
        out_shape=[
            jax.ShapeDtypeStruct((N_NODES, D), jnp.float32),
            jax.ShapeDtypeStruct((N_NODES, D), jnp.float32),
        ],
    )(x, W1, b1.reshape(1, D))


def _sc_body(src_hbm, dst_hbm, emb_hbm, a_hbm, b_hbm, batch_hbm,
             xb_out, aggb_out,
             srcv, gdstv, cidv, arows, brows, bidxv,
             markv, batchv, srcsh, dstsh,
             aggsh, sem1, sem2):
    cid = lax.axis_index("c")
    sid = lax.axis_index("s")
    wid = sid * NC + cid

    zero16 = jnp.zeros((VL,), jnp.float32)
    ones16 = jnp.ones((VL,), jnp.int32)

    def zrow(i, carry):
        for j in range(D // VL):
            arows[i, pl.ds(j * VL, VL)] = zero16
        return carry

    lax.fori_loop(0, K, zrow, None)
    z0 = sid * ZR
    pltpu.sync_copy(arows, aggsh.at[pl.ds(z0, K)])
    pltpu.sync_copy(arows, aggsh.at[pl.ds(z0 + K, K)])
    pltpu.sync_copy(arows.at[pl.ds(0, ZR - 2 * K)], aggsh.at[pl.ds(z0 + 2 * K, ZR - 2 * K)])

    def zmark(i, carry):
        markv[pl.ds(i * VL, VL)] = jnp.zeros((VL,), jnp.int32)
        return carry

    lax.fori_loop(0, NMARK // VL, zmark, None)
    pltpu.sync_copy(batch_hbm, batchv)

    def scat(i, carry):
        idx = batchv[pl.ds(i * VL, VL)]
        plsc.store_scatter(markv, [idx], ones16)
        return carry

    lax.fori_loop(0, BATCH // VL, scat, None)

    def scan(i, carry):
        s = pl.ds(i * VL, VL)
        f = markv[s]
        ids = carry + plsc.cumsum(f) - 1
        markv[s] = jnp.where(f > 0, ids, jnp.full((VL,), GARBAGE, jnp.int32))
        return carry + jnp.sum(f)

    lax.fori_loop(0, NMARK // VL, scan, jnp.int32(0))

    pltpu.sync_copy(src_hbm.at[pl.ds(wid * EPW, EPW)], srcsh.at[pl.ds(0, EPW)])
    pltpu.sync_copy(dst_hbm.at[pl.ds(wid * EPW, EPW)], dstsh.at[pl.ds(0, EPW)])

    def compress(i, cur):
        s = pl.ds(i * VL, VL)
        d = dstsh[s]
        sv = srcsh[s]
        keep = plsc.load_gather(markv, [d]) != GARBAGE
        plsc.store_compressed(dstsh.at[pl.ds(cur, VL)], d, mask=keep)
        plsc.store_compressed(srcsh.at[pl.ds(cur, VL)], sv, mask=keep)
        return cur + jnp.sum(keep.astype(jnp.int32))

    cnt = lax.fori_loop(0, EPW // VL, compress, jnp.int32(0))

    pad_dst = jnp.full((VL,), N_NODES, jnp.int32)
    pad_src = jnp.zeros((VL,), jnp.int32)

    def padk(i, carry):
        dstsh[pl.ds(cnt + i * VL, VL)] = pad_dst
        srcsh[pl.ds(cnt + i * VL, VL)] = pad_src
        return carry

    lax.fori_loop(0, K // VL, padk, None)
    nchunks = (cnt + K - 1) >> 7

    plsc.subcore_barrier()

    def edge_chunk(c, carry):
        for j in range(K // VL):
            s = pl.ds(j * VL, VL)
            dv = dstsh[pl.ds(c * K + j * VL, VL)]
            srcv[s] = srcsh[pl.ds(c * K + j * VL, VL)]
            gdstv[s] = jnp.minimum(dv, N_NODES - 1)
            cidv[s] = plsc.load_gather(markv, [dv])
        cpa = pltpu.async_copy(a_hbm.at[srcv], arows, sem1)
        cpb = pltpu.async_copy(b_hbm.at[gdstv], brows, sem2)
        cpa.wait()
        cpb.wait()

        def fuse(i, inner):
            for j in range(D // VL):
                s = pl.ds(j * VL, VL)
                arows[i, s] = jnp.maximum(arows[i, s] + brows[i, s], 0.0)
            return inner

        lax.fori_loop(0, K, fuse, None)
        pltpu.sync_copy(arows, aggsh.at[cidv], add=True)
        return carry

    lax.fori_loop(0, nchunks, edge_chunk, None)
    plsc.subcore_barrier()

    base = wid * BPW
    for j in range(BPW // VL):
        bidxv[pl.ds(j * VL, VL)] = batchv[pl.ds(base + j * VL, VL)]
    pltpu.async_copy(emb_hbm.at[bidxv], arows, sem1).wait()
    pltpu.sync_copy(arows, xb_out.at[pl.ds(base, BPW)])

    for r in range(BPT // BPW):
        b0 = sid * BPT + r * BPW
        for j in range(BPW // VL):
            bv = batchv[pl.ds(b0 + j * VL, VL)]
            bidxv[pl.ds(j * VL, VL)] = plsc.load_gather(markv, [bv])
        pltpu.async_copy(aggsh.at[bidxv], arows, sem2).wait()
        pltpu.sync_copy(arows, aggb_out.at[cid, pl.ds(b0, BPW)])


_sc_call = pl.kernel(
    _sc_body,
    out_type=(
        jax.ShapeDtypeStruct((BATCH, D), jnp.float32),
        jax.ShapeDtypeStruct((NC, BATCH, D), jnp.float32),
    ),
    mesh=plsc.VectorSubcoreMesh(core_axis_name="c", subcore_axis_name="s"),
    scratch_types=[
        pltpu.VMEM((K,), jnp.int32),
        pltpu.VMEM((K,), jnp.int32),
        pltpu.VMEM((K,), jnp.int32),
        pltpu.VMEM((K, D), jnp.float32),
        pltpu.VMEM((K, D), jnp.float32),
        pltpu.VMEM((BPW,), jnp.int32),
        pltpu.VMEM((NMARK,), jnp.int32),
        pltpu.VMEM((BATCH,), jnp.int32),
        pltpu.VMEM((EPW + K,), jnp.int32),
        pltpu.VMEM((EPW + K,), jnp.int32),
        pltpu.VMEM_SHARED((ACC, D), jnp.float32),
        pltpu.SemaphoreType.DMA,
        pltpu.SemaphoreType.DMA,
    ],
    compiler_params=pltpu.CompilerParams(needs_layout_passes=False),
)


def _tc2_body(xb_ref, a0_ref, a1_ref, w_ref, b_ref, o_ref):
    o_ref[...] = (
        jnp.dot(xb_ref[...], w_ref[:D, :], preferred_element_type=jnp.float32)
        + jnp.dot(a0_ref[...] + a1_ref[...], w_ref[D:, :],
                  preferred_element_type=jnp.float32)
        + b_ref[...]
    )


def _final(xb, a0, a1, W2, b2):
    blk = 1024
    return pl.pallas_call(
        _tc2_body,
        grid=(BATCH // blk,),
        in_specs=[
            pl.BlockSpec((blk, D), lambda i: (i, 0)),
            pl.BlockSpec((blk, D), lambda i: (i, 0)),
            pl.BlockSpec((blk, D), lambda i: (i, 0)),
            pl.BlockSpec((2 * D, D), lambda i: (0, 0)),
            pl.BlockSpec((1, D), lambda i: (0, 0)),
        ],
        out_specs=pl.BlockSpec((blk, D), lambda i: (i, 0)),
        out_shape=jax.ShapeDtypeStruct((BATCH, D), jnp.float32),
    )(xb, a0, a1, W2, b2.reshape(1, D))


def kernel(batch, edge_index, emb, W1, b1, W2, b2):
    a, bb = _precompute_ab(emb, W1, b1)
    xb, aggb = _sc_call(edge_index[0], edge_index[1], emb, a, bb, batch)
    return _final(xb, aggb[0], aggb[1], W2, b2)


# speedup vs baseline: 11.8577x; 1.0892x over previous
import functools

import jax
import jax.numpy as jnp
from jax import lax
from jax.experimental import pallas as pl
from jax.experimental.pallas import tpu as pltpu
from jax.experimental.pallas import tpu_sc as plsc

N_NODES = 10000
D = 128
E = 320000
BATCH = 4096

NC, NS = 2, 16
NW = NC * NS
EPW = E // NW
K = 128
BPW = BATCH // NW
BPT = BATCH // NS
GARBAGE = BATCH
ACC = 4112
ZR = ACC // NS
VL = 16
NMARK = N_NODES + VL


def _tc1_body(x_ref, w_ref, b_ref, a_ref, bb_ref):
    x = x_ref[...]
    a_ref[...] = jnp.dot(x, w_ref[:D, :], preferred_element_type=jnp.float32)
    bb_ref[...] = (
        jnp.dot(x, w_ref[D:, :], preferred_element_type=jnp.float32) + b_ref[...]
    )


def _precompute_ab(x, W1, b1):
    blk = N_NODES // 10
    return pl.pallas_call(
        _tc1_body,
        grid=(N_NODES // blk,),
        in_specs=[
            pl.BlockSpec((blk, D), lambda i: (i, 0)),
            pl.BlockSpec((2 * D, D), lambda i: (0, 0)),
            pl.BlockSpec((1, D), lambda i: (0, 0)),
        ],
        out_specs=[
            pl.BlockSpec((blk, D), lambda i: (i, 0)),
            pl.BlockSpec((blk, D), lambda i: (i, 0)),
        ],
        out_shape=[
            jax.ShapeDtypeStruct((N_NODES, D), jnp.float32),
            jax.ShapeDtypeStruct((N_NODES, D), jnp.float32),
        ],
    )(x, W1, b1.reshape(1, D))


def _sc_body(src_hbm, dst_hbm, emb_hbm, a_hbm, b_hbm, batch_hbm,
             xb_out, aggb_out,
             srcv0, gdstv0, cidv0, arows0, brows0,
             srcv1, gdstv1, cidv1, arows1, brows1,
             bidxv, markv, srcsh, dstsh,
             aggsh, sa0, sb0, sa1, sb1):
    cid = lax.axis_index("c")
    sid = lax.axis_index("s")
    wid = sid * NC + cid

    zero16 = jnp.zeros((VL,), jnp.float32)
    ones16 = jnp.ones((VL,), jnp.int32)

    def zrow(i, carry):
        for j in range(D // VL):
            arows0[i, pl.ds(j * VL, VL)] = zero16
        return carry

    lax.fori_loop(0, K, zrow, None)
    z0 = sid * ZR
    pltpu.sync_copy(arows0, aggsh.at[pl.ds(z0, K)])
    pltpu.sync_copy(arows0, aggsh.at[pl.ds(z0 + K, K)])
    pltpu.sync_copy(arows0.at[pl.ds(0, ZR - 2 * K)],
                    aggsh.at[pl.ds(z0 + 2 * K, ZR - 2 * K)])

    def zmark(i, carry):
        markv[pl.ds(i * VL, VL)] = jnp.zeros((VL,), jnp.int32)
        return carry

    lax.fori_loop(0, NMARK // VL, zmark, None)

    for ch in range(BATCH // BPW):
        pltpu.sync_copy(batch_hbm.at[pl.ds(ch * BPW, BPW)], bidxv)

        def scat(i, carry):
            idx = bidxv[pl.ds(i * VL, VL)]
            plsc.store_scatter(markv, [idx], ones16)
            return carry

        lax.fori_loop(0, BPW // VL, scat, None)

    def scan(i, carry):
        s = pl.ds(i * VL, VL)
        f = markv[s]
        ids = carry + plsc.cumsum(f) - 1
        markv[s] = jnp.where(f > 0, ids, jnp.full((VL,), GARBAGE, jnp.int32))
        return carry + jnp.sum(f)

    lax.fori_loop(0, NMARK // VL, scan, jnp.int32(0))

    pltpu.sync_copy(src_hbm.at[pl.ds(wid * EPW, EPW)], srcsh.at[pl.ds(0, EPW)])
    pltpu.sync_copy(dst_hbm.at[pl.ds(wid * EPW, EPW)], dstsh.at[pl.ds(0, EPW)])

    def compress(i, cur):
        s = pl.ds(i * VL, VL)
        d = dstsh[s]
        sv = srcsh[s]
        keep = plsc.load_gather(markv, [d]) != GARBAGE
        plsc.store_compressed(dstsh.at[pl.ds(cur, VL)], d, mask=keep)
        plsc.store_compressed(srcsh.at[pl.ds(cur, VL)], sv, mask=keep)
        return cur + jnp.sum(keep.astype(jnp.int32))

    cnt = lax.fori_loop(0, EPW // VL, compress, jnp.int32(0))

    pad_dst = jnp.full((VL,), N_NODES, jnp.int32)
    pad_src = jnp.zeros((VL,), jnp.int32)

    def padk(i, carry):
        dstsh[pl.ds(cnt + i * VL, VL)] = pad_dst
        srcsh[pl.ds(cnt + i * VL, VL)] = pad_src
        return carry

    lax.fori_loop(0, K // VL, padk, None)
    nchunks = (cnt + K - 1) >> 7

    plsc.subcore_barrier()

    sets = ((srcv0, gdstv0, cidv0, arows0, brows0, sa0, sb0),
            (srcv1, gdstv1, cidv1, arows1, brows1, sa1, sb1))

    def build_issue(cc, p):
        sv, gv, cv, ar, br, sa, sb = sets[p]
        for j in range(K // VL):
            s = pl.ds(j * VL, VL)
            dv = dstsh[pl.ds(cc * K + j * VL, VL)]
            sv[s] = srcsh[pl.ds(cc * K + j * VL, VL)]
            gv[s] = jnp.minimum(dv, N_NODES - 1)
            cv[s] = plsc.load_gather(markv, [dv])
        pltpu.async_copy(a_hbm.at[sv], ar, sa)
        pltpu.async_copy(b_hbm.at[gv], br, sb)

    def step(c, p):
        sv, gv, cv, ar, br, sa, sb = sets[p]
        pltpu.make_async_copy(a_hbm.at[sv], ar, sa).wait()
        pltpu.make_async_copy(b_hbm.at[gv], br, sb).wait()

        @pl.when(c + 1 < nchunks)
        def _():
            build_issue(c + 1, 1 - p)

        def fuse(i, inner):
            for j in range(D // VL):
                s = pl.ds(j * VL, VL)
                ar[i, s] = jnp.maximum(ar[i, s] + br[i, s], 0.0)
            return inner

        lax.fori_loop(0, K, fuse, None)
        pltpu.sync_copy(ar, aggsh.at[cv], add=True)

    @pl.when(nchunks > 0)
    def _():
        build_issue(0, 0)

    def pair(c2, carry):
        c = c2 * 2

        @pl.when(c < nchunks)
        def _():
            step(c, 0)

        @pl.when(c + 1 < nchunks)
        def _():
            step(c + 1, 1)

        return carry

    lax.fori_loop(0, (nchunks + 1) >> 1, pair, None)
    plsc.subcore_barrier()

    base = wid * BPW
    pltpu.sync_copy(batch_hbm.at[pl.ds(base, BPW)], bidxv)
    pltpu.async_copy(emb_hbm.at[bidxv], arows0, sa0).wait()
    pltpu.sync_copy(arows0, xb_out.at[pl.ds(base, BPW)])

    for r in range(BPT // BPW):
        b0 = sid * BPT + r * BPW
        pltpu.sync_copy(batch_hbm.at[pl.ds(b0, BPW)], bidxv)
        for j in range(BPW // VL):
            s = pl.ds(j * VL, VL)
            bidxv[s] = plsc.load_gather(markv, [bidxv[s]])
        pltpu.async_copy(aggsh.at[bidxv], arows0, sb0).wait()
        pltpu.sync_copy(arows0, aggb_out.at[cid, pl.ds(b0, BPW)])


_sc_call = pl.kernel(
    _sc_body,
    out_type=(
        jax.ShapeDtypeStruct((BATCH, D), jnp.float32),
        jax.ShapeDtypeStruct((NC, BATCH, D), jnp.float32),
    ),
    mesh=plsc.VectorSubcoreMesh(core_axis_name="c", subcore_axis_name="s"),
    scratch_types=[
        pltpu.VMEM((K,), jnp.int32),
        pltpu.VMEM((K,), jnp.int32),
        pltpu.VMEM((K,), jnp.int32),
        pltpu.VMEM((K, D), jnp.float32),
        pltpu.VMEM((K, D), jnp.float32),
        pltpu.VMEM((K,), jnp.int32),
        pltpu.VMEM((K,), jnp.int32),
        pltpu.VMEM((K,), jnp.int32),
        pltpu.VMEM((K, D), jnp.float32),
        pltpu.VMEM((K, D), jnp.float32),
        pltpu.VMEM((BPW,), jnp.int32),
        pltpu.VMEM((NMARK,), jnp.int32),
        pltpu.VMEM((EPW + K,), jnp.int32),
        pltpu.VMEM((EPW + K,), jnp.int32),
        pltpu.VMEM_SHARED((ACC, D), jnp.float32),
        pltpu.SemaphoreType.DMA,
        pltpu.SemaphoreType.DMA,
        pltpu.SemaphoreType.DMA,
        pltpu.SemaphoreType.DMA,
    ],
    compiler_params=pltpu.CompilerParams(needs_layout_passes=False),
)


def _tc2_body(xb_ref, a0_ref, a1_ref, w_ref, b_ref, o_ref):
    o_ref[...] = (
        jnp.dot(xb_ref[...], w_ref[:D, :], preferred_element_type=jnp.float32)
        + jnp.dot(a0_ref[...] + a1_ref[...], w_ref[D:, :],
                  preferred_element_type=jnp.float32)
        + b_ref[...]
    )


def _final(xb, a0, a1, W2, b2):
    blk = 1024
    return pl.pallas_call(
        _tc2_body,
        grid=(BATCH // blk,),
        in_specs=[
            pl.BlockSpec((blk, D), lambda i: (i, 0)),
            pl.BlockSpec((blk, D), lambda i: (i, 0)),
            pl.BlockSpec((blk, D), lambda i: (i, 0)),
            pl.BlockSpec((2 * D, D), lambda i: (0, 0)),
            pl.BlockSpec((1, D), lambda i: (0, 0)),
        ],
        out_specs=pl.BlockSpec((blk, D), lambda i: (i, 0)),
        out_shape=jax.ShapeDtypeStruct((BATCH, D), jnp.float32),
    )(xb, a0, a1, W2, b2.reshape(1, D))


def kernel(batch, edge_index, emb, W1, b1, W2, b2):
    a, bb = _precompute_ab(emb, W1, b1)
    xb, aggb = _sc_call(edge_index[0], edge_index[1], emb, a, bb, batch)
    return _final(xb, aggb[0], aggb[1], W2, b2)
